# Initial kernel scaffold; baseline (speedup 1.0000x reference)
import functools

import jax
import jax.numpy as jnp
from jax import lax
from jax.experimental import pallas as pl
from jax.experimental.pallas import tpu as pltpu
from jax.experimental.pallas import tpu_sc as plsc

NC = 2
NS = 16
NW = NC * NS

B_ROWS = 1600
L = 128
D = 128
ROWS_PER_W = B_ROWS // NW


def _gather_body(table_hbm, idx_hbm, out_hbm, idx_v, buf, sem):
    wid = lax.axis_index("s") * NC + lax.axis_index("c")
    base = wid * ROWS_PER_W
    pltpu.sync_copy(idx_hbm.at[pl.ds(base, ROWS_PER_W)], idx_v)

    def step(j, carry):
        pltpu.async_copy(table_hbm.at[idx_v.at[j]], buf, sem).wait()
        pltpu.sync_copy(buf, out_hbm.at[pl.ds((base + j) * L, L)])
        return carry

    lax.fori_loop(0, ROWS_PER_W, step, 0)


@jax.jit
def _lookup(table, idx):
    mesh = plsc.VectorSubcoreMesh(core_axis_name="c", subcore_axis_name="s")
    call = functools.partial(
        pl.kernel,
        mesh=mesh,
        out_type=jax.ShapeDtypeStruct((B_ROWS * L, D), jnp.float32),
        scratch_types=[
            pltpu.VMEM((ROWS_PER_W, L), jnp.int32),
            pltpu.VMEM((L, D), jnp.float32),
            pltpu.SemaphoreType.DMA,
        ],
    )(_gather_body)
    return call(table, idx)


def kernel(time_features, table):
    table = table.at[0].set(0.0)
    idx = time_features.reshape(B_ROWS, L)
    out = _lookup(table, idx)
    return out.reshape(4096, 50, D)


# speedup vs baseline: 2.7428x; 2.7428x over previous
import functools

import jax
import jax.numpy as jnp
from jax import lax
from jax.experimental import pallas as pl
from jax.experimental.pallas import tpu as pltpu
from jax.experimental.pallas import tpu_sc as plsc

NC = 2
NS = 16
NW = NC * NS

B_ROWS = 1600
L = 128
D = 128
ROWS_PER_W = B_ROWS // NW


def _gather_body(table_hbm, idx_hbm, out_hbm, idx_v, buf, sem):
    wid = lax.axis_index("s") * NC + lax.axis_index("c")
    base = wid * ROWS_PER_W
    pltpu.sync_copy(idx_hbm.at[wid], idx_v)

    def step(j, carry):
        pltpu.async_copy(table_hbm.at[idx_v.at[j]], buf, sem).wait()
        pltpu.sync_copy(buf, out_hbm.at[pl.ds((base + j) * L, L)])
        return carry

    lax.fori_loop(0, ROWS_PER_W, step, 0)


@jax.jit
def _lookup(table, idx):
    mesh = plsc.VectorSubcoreMesh(core_axis_name="c", subcore_axis_name="s")
    call = functools.partial(
        pl.kernel,
        mesh=mesh,
        out_type=jax.ShapeDtypeStruct((B_ROWS * L, D), jnp.float32),
        scratch_types=[
            pltpu.VMEM((ROWS_PER_W, L), jnp.int32),
            pltpu.VMEM((L, D), jnp.float32),
            pltpu.SemaphoreType.DMA,
        ],
    )(_gather_body)
    return call(table, idx)


def kernel(time_features, table):
    table = table.at[0].set(0.0)
    idx = time_features.reshape(NW, ROWS_PER_W, L)
    out = _lookup(table, idx)
    return out.reshape(4096, 50, D)


# speedup vs baseline: 3.1141x; 1.1353x over previous
import functools

import jax
import jax.numpy as jnp
from jax import lax
from jax.experimental import pallas as pl
from jax.experimental.pallas import tpu as pltpu
from jax.experimental.pallas import tpu_sc as plsc

NC = 2
NS = 16
NW = NC * NS

B_ROWS = 1600
L = 128
D = 128
ROWS_PER_W = B_ROWS // NW


NBUF = 5


def _gather_body(table_hbm, idx_hbm, out_hbm, idx_v, buf,
                 sg0, sg1, sg2, sg3, sg4):
    sg = (sg0, sg1, sg2, sg3, sg4)
    wid = lax.axis_index("s") * NC + lax.axis_index("c")
    base = wid * ROWS_PER_W
    pltpu.sync_copy(idx_hbm.at[wid], idx_v)

    for b in range(NBUF):
        pltpu.async_copy(table_hbm.at[idx_v.at[b]], buf.at[b], sg[b])

    def outer(i, carry):
        j0 = i * NBUF
        for b in range(NBUF):
            j = j0 + b
            pltpu.make_async_copy(
                table_hbm.at[idx_v.at[j]], buf.at[b], sg[b]).wait()
            pltpu.sync_copy(buf.at[b], out_hbm.at[pl.ds((base + j) * L, L)])
            pltpu.async_copy(
                table_hbm.at[idx_v.at[j + NBUF]], buf.at[b], sg[b])
        return carry

    lax.fori_loop(0, ROWS_PER_W // NBUF - 1, outer, 0)

    for b in range(NBUF):
        j = ROWS_PER_W - NBUF + b
        pltpu.make_async_copy(
            table_hbm.at[idx_v.at[j]], buf.at[b], sg[b]).wait()
        pltpu.sync_copy(buf.at[b], out_hbm.at[pl.ds((base + j) * L, L)])


@jax.jit
def _lookup(table, idx):
    mesh = plsc.VectorSubcoreMesh(core_axis_name="c", subcore_axis_name="s")
    call = functools.partial(
        pl.kernel,
        mesh=mesh,
        out_type=jax.ShapeDtypeStruct((B_ROWS * L, D), jnp.float32),
        scratch_types=[
            pltpu.VMEM((ROWS_PER_W, L), jnp.int32),
            pltpu.VMEM((NBUF, L, D), jnp.float32),
        ] + [pltpu.SemaphoreType.DMA] * NBUF,
    )(_gather_body)
    return call(table, idx)


def kernel(time_features, table):
    table = table.at[0].set(0.0)
    idx = time_features.reshape(NW, ROWS_PER_W, L)
    out = _lookup(table, idx)
    return out.reshape(4096, 50, D)
